# Initial kernel scaffold; baseline (speedup 1.0000x reference)
import jax
import jax.numpy as jnp
from jax.experimental import pallas as pl


def kernel(year, answer, answer_table, year_table, question_table, question_range):
    raise NotImplementedError("write your pallas kernel here")


# speedup vs baseline: 9.7016x; 9.7016x over previous
import jax
import jax.numpy as jnp
from jax import lax
from jax.experimental import pallas as pl
from jax.experimental.pallas import tpu as pltpu
from jax.experimental.pallas import tpu_sc as plsc

B = 16384
VOCAB = 1000
NQ = 100
NY = 14
D = 64

NC = 2
NS = 16
L = 16
NW = NC * NS
BPW = B // NW
CB = 4
K = BPW // CB
ROWS = CB * NQ
NCH = D // L


def _sc_body(year_hbm, ans_hbm, table_hbm, qtab_hbm, ytab_hbm, out_hbm,
             year_v, q_v,
             idx0, idx1, yr0, yr1, in0, in1, ob0, ob1,
             sg0, sg1, so0, so1):
    cid = lax.axis_index("c")
    sid = lax.axis_index("s")
    wid = sid * NC + cid
    b0 = wid * BPW

    idx_bufs = (idx0, idx1)
    yrow_bufs = (yr0, yr1)
    in_bufs = (in0, in1)
    out_bufs = (ob0, ob1)
    sg = (sg0, sg1)
    so = (so0, so1)

    pltpu.sync_copy(year_hbm.at[pl.ds(wid * K, K)], year_v)
    pltpu.sync_copy(qtab_hbm, q_v)

    def issue_in(k, slot):
        pltpu.sync_copy(ans_hbm.at[pl.ds(b0 + k * CB, CB)], idx_bufs[slot])
        for j in range(CB):
            pltpu.async_copy(table_hbm.at[idx_bufs[slot].at[j]],
                             in_bufs[slot].at[pl.ds(j * NQ, NQ)], sg[slot])
        pltpu.async_copy(ytab_hbm.at[year_v.at[k]],
                         yrow_bufs[slot], sg[slot])

    def wait_in(k, slot):
        for j in range(CB):
            pltpu.make_async_copy(table_hbm.at[idx_bufs[slot].at[j]],
                                  in_bufs[slot].at[pl.ds(j * NQ, NQ)],
                                  sg[slot]).wait()
        pltpu.make_async_copy(ytab_hbm.at[year_v.at[k]],
                              yrow_bufs[slot], sg[slot]).wait()

    def issue_out(k, slot):
        rowbase = (b0 + k * CB) * NQ
        pltpu.async_copy(out_bufs[slot], out_hbm.at[pl.ds(rowbase, ROWS)],
                         so[slot])

    def wait_out(slot):
        pltpu.make_async_copy(out_bufs[slot], out_hbm.at[pl.ds(0, ROWS)],
                              so[slot]).wait()

    def compute(k, slot):
        inb = in_bufs[slot]
        outb = out_bufs[slot]
        yrb = yrow_bufs[slot]
        for jb in range(CB):
            yrows = [yrb[jb, pl.ds(16 * c, 16)] for c in range(NCH)]

            def qbody(q, _, jb=jb, yrows=yrows):
                r = jb * NQ + q
                for c in range(NCH):
                    v = (inb[r, pl.ds(16 * c, 16)]
                         + q_v[q, pl.ds(16 * c, 16)] + yrows[c])
                    outb[r, pl.ds(16 * c, 16)] = v
                return 0

            lax.fori_loop(0, NQ, qbody, 0)

    for s in range(2):
        issue_in(s, s)

    for s in range(2):
        wait_in(s, s)
        compute(s, s)
        issue_out(s, s)
        issue_in(s + 2, s)

    def outer(kk, _):
        for s in range(2):
            k = kk * 2 + s
            wait_in(k, s)
            wait_out(s)
            compute(k, s)
            issue_out(k, s)
            issue_in(k + 2, s)
        return 0

    lax.fori_loop(1, K // 2 - 1, outer, 0)

    for s in range(2):
        k = K - 2 + s
        wait_in(k, s)
        wait_out(s)
        compute(k, s)
        issue_out(k, s)
    for s in range(2):
        wait_out(s)


def kernel(year, answer, answer_table, year_table, question_table,
           question_range):
    year = jnp.asarray(year, jnp.int32).reshape(B // CB, CB)
    answer = jnp.asarray(answer, jnp.int32)
    qtab = jnp.take(question_table, question_range, axis=0)

    mesh = plsc.VectorSubcoreMesh(core_axis_name="c", subcore_axis_name="s",
                                  num_cores=NC, num_subcores=NS)
    run = pl.kernel(
        _sc_body,
        out_type=jax.ShapeDtypeStruct((B * NQ, D), jnp.float32),
        mesh=mesh,
        compiler_params=pltpu.CompilerParams(needs_layout_passes=False,
                                             use_tc_tiling_on_sc=False),
        scratch_types=[
            pltpu.VMEM((K, CB), jnp.int32),
            pltpu.VMEM((NQ, D), jnp.float32),
            pltpu.VMEM((CB, NQ), jnp.int32),
            pltpu.VMEM((CB, NQ), jnp.int32),
            pltpu.VMEM((CB, D), jnp.float32),
            pltpu.VMEM((CB, D), jnp.float32),
            pltpu.VMEM((ROWS, D), jnp.float32),
            pltpu.VMEM((ROWS, D), jnp.float32),
            pltpu.VMEM((ROWS, D), jnp.float32),
            pltpu.VMEM((ROWS, D), jnp.float32),
            pltpu.SemaphoreType.DMA,
            pltpu.SemaphoreType.DMA,
            pltpu.SemaphoreType.DMA,
            pltpu.SemaphoreType.DMA,
        ],
    )
    out = run(year, answer, answer_table, qtab, year_table)
    return out.reshape(B, NQ, D)


# speedup vs baseline: 9.7144x; 1.0013x over previous
import jax
import jax.numpy as jnp
from jax import lax
from jax.experimental import pallas as pl
from jax.experimental.pallas import tpu as pltpu
from jax.experimental.pallas import tpu_sc as plsc

B = 16384
VOCAB = 1000
NQ = 100
NY = 14
D = 64

NC = 2
NS = 16
L = 16
NW = NC * NS
BPW = B // NW
CB = 4
K = BPW // CB
ROWS = CB * NQ
NCH = D // L


def _sc_body(year_hbm, ans_hbm, table_hbm, qtab_hbm, ytab_hbm, out_hbm,
             year_v, q_v,
             idx0, idx1, yr0, yr1, in0, in1, ob0, ob1,
             sg0, sg1, so0, so1):
    cid = lax.axis_index("c")
    sid = lax.axis_index("s")
    wid = sid * NC + cid
    b0 = wid * BPW

    idx_bufs = (idx0, idx1)
    yrow_bufs = (yr0, yr1)
    in_bufs = (in0, in1)
    out_bufs = (ob0, ob1)
    sg = (sg0, sg1)
    so = (so0, so1)

    pltpu.sync_copy(year_hbm.at[pl.ds(wid * K, K)], year_v)
    pltpu.sync_copy(qtab_hbm, q_v)

    def issue_in(k, slot):
        pltpu.sync_copy(ans_hbm.at[pl.ds(b0 + k * CB, CB)], idx_bufs[slot])
        for j in range(CB):
            pltpu.async_copy(table_hbm.at[idx_bufs[slot].at[j]],
                             in_bufs[slot].at[pl.ds(j * NQ, NQ)], sg[slot])
        pltpu.async_copy(ytab_hbm.at[year_v.at[k]],
                         yrow_bufs[slot], sg[slot])

    def wait_in(k, slot):
        for j in range(CB):
            pltpu.make_async_copy(table_hbm.at[idx_bufs[slot].at[j]],
                                  in_bufs[slot].at[pl.ds(j * NQ, NQ)],
                                  sg[slot]).wait()
        pltpu.make_async_copy(ytab_hbm.at[year_v.at[k]],
                              yrow_bufs[slot], sg[slot]).wait()

    def issue_out(k, slot):
        rowbase = (b0 + k * CB) * NQ // 2
        pltpu.async_copy(out_bufs[slot], out_hbm.at[pl.ds(rowbase, ROWS // 2)],
                         so[slot])

    def wait_out(slot):
        pltpu.make_async_copy(out_bufs[slot], out_hbm.at[pl.ds(0, ROWS // 2)],
                              so[slot]).wait()

    def compute(k, slot):
        inb = in_bufs[slot]
        outb = out_bufs[slot]
        yrb = yrow_bufs[slot]
        for jb in range(CB):
            yrows = [yrb[jb, pl.ds(16 * c, 16)] for c in range(NCH)]

            def qbody(q2, _, jb=jb, yrows=yrows):
                for dq in range(2):
                    q = 2 * q2 + dq
                    r = jb * NQ + q
                    for c in range(NCH):
                        v = (inb[r, pl.ds(16 * c, 16)]
                             + q_v[q, pl.ds(16 * c, 16)] + yrows[c])
                        outb[jb * (NQ // 2) + q2,
                             pl.ds(64 * dq + 16 * c, 16)] = v
                return 0

            lax.fori_loop(0, NQ // 2, qbody, 0)

    for s in range(2):
        issue_in(s, s)

    for s in range(2):
        wait_in(s, s)
        compute(s, s)
        issue_out(s, s)
        issue_in(s + 2, s)

    def outer(kk, _):
        for s in range(2):
            k = kk * 2 + s
            wait_in(k, s)
            wait_out(s)
            compute(k, s)
            issue_out(k, s)
            issue_in(k + 2, s)
        return 0

    lax.fori_loop(1, K // 2 - 1, outer, 0)

    for s in range(2):
        k = K - 2 + s
        wait_in(k, s)
        wait_out(s)
        compute(k, s)
        issue_out(k, s)
    for s in range(2):
        wait_out(s)


def kernel(year, answer, answer_table, year_table, question_table,
           question_range):
    year = jnp.asarray(year, jnp.int32).reshape(B // CB, CB)
    answer = jnp.asarray(answer, jnp.int32)
    qtab = jnp.take(question_table, question_range, axis=0)

    mesh = plsc.VectorSubcoreMesh(core_axis_name="c", subcore_axis_name="s",
                                  num_cores=NC, num_subcores=NS)
    run = pl.kernel(
        _sc_body,
        out_type=jax.ShapeDtypeStruct((B * NQ // 2, 2 * D), jnp.float32),
        mesh=mesh,
        compiler_params=pltpu.CompilerParams(needs_layout_passes=False,
                                             use_tc_tiling_on_sc=False),
        scratch_types=[
            pltpu.VMEM((K, CB), jnp.int32),
            pltpu.VMEM((NQ, D), jnp.float32),
            pltpu.VMEM((CB, NQ), jnp.int32),
            pltpu.VMEM((CB, NQ), jnp.int32),
            pltpu.VMEM((CB, D), jnp.float32),
            pltpu.VMEM((CB, D), jnp.float32),
            pltpu.VMEM((ROWS, D), jnp.float32),
            pltpu.VMEM((ROWS, D), jnp.float32),
            pltpu.VMEM((ROWS // 2, 2 * D), jnp.float32),
            pltpu.VMEM((ROWS // 2, 2 * D), jnp.float32),
            pltpu.SemaphoreType.DMA,
            pltpu.SemaphoreType.DMA,
            pltpu.SemaphoreType.DMA,
            pltpu.SemaphoreType.DMA,
        ],
    )
    out = run(year, answer, answer_table, qtab, year_table)
    return out.reshape(B, NQ, D)
